# baseline (speedup 1.0000x reference)
import functools
import jax
import jax.numpy as jnp
from jax import lax
from jax.experimental import pallas as pl
from jax.experimental.pallas import tpu as pltpu
from jax.experimental.pallas import tpu_sc as plsc

N = 10000
E = 160000
EMBED = 128
HID = 64
VEC_OUT = 64

NC = 2
NS = 16
EP = 163840
NP = 10240
B3 = 80
B1 = 512

_HIGH = jax.lax.Precision.HIGHEST


def _p1_body(pos_hbm, snd_hbm, rcv_hbm, zeros16_hbm, ones_hbm,
             relx_hbm, rely_hbm, relz_hbm, cnt_hbm,
             pos_v, snd_v, rcv_v, ox_v, oy_v, oz_v, ones_v, cnt_sh):
    core = lax.axis_index("c")
    sub = lax.axis_index("s")
    wid = sub * NC + core
    rows_per_sub = NP // NS
    pltpu.sync_copy(pos_hbm, pos_v)
    pltpu.sync_copy(ones_hbm, ones_v)
    for j in range(rows_per_sub // 64):
        pltpu.sync_copy(zeros16_hbm, cnt_sh.at[pl.ds(sub * rows_per_sub + j * 64, 64)])
    plsc.subcore_barrier()
    per_tile = EP // (NC * NS)

    def batch(b, _):
        e0 = wid * per_tile + b * B1
        pltpu.sync_copy(snd_hbm.at[pl.ds(e0, B1)], snd_v)
        pltpu.sync_copy(rcv_hbm.at[pl.ds(e0, B1)], rcv_v)
        for g in range(B1 // 16):
            sl = pl.ds(g * 16, 16)
            s3 = snd_v[sl] * 3
            r3 = rcv_v[sl] * 3
            for d, ref in ((0, ox_v), (1, oy_v), (2, oz_v)):
                ps = plsc.load_gather(pos_v, [s3 + d])
                pr = plsc.load_gather(pos_v, [r3 + d])
                ref[sl] = pr - ps
        pltpu.sync_copy(ox_v, relx_hbm.at[pl.ds(e0, B1)])
        pltpu.sync_copy(oy_v, rely_hbm.at[pl.ds(e0, B1)])
        pltpu.sync_copy(oz_v, relz_hbm.at[pl.ds(e0, B1)])
        pltpu.sync_copy(ones_v, cnt_sh.at[rcv_v], add=True)
        return ()

    lax.fori_loop(0, per_tile // B1, batch, ())
    plsc.subcore_barrier()
    r0 = sub * rows_per_sub
    pltpu.sync_copy(cnt_sh.at[pl.ds(r0, rows_per_sub)],
                    cnt_hbm.at[core, pl.ds(r0, rows_per_sub)])


def _p1_call(pos_flat, snd, rcv):
    zeros16 = jnp.zeros((64, 16), jnp.float32)
    ones = jnp.ones((B1, 16), jnp.float32)
    return pl.kernel(
        _p1_body,
        out_type=(jax.ShapeDtypeStruct((EP,), jnp.float32),) * 3
                 + (jax.ShapeDtypeStruct((2, NP, 16), jnp.float32),),
        mesh=plsc.VectorSubcoreMesh(core_axis_name="c", subcore_axis_name="s"),
        scratch_types=[
            pltpu.VMEM((N * 3,), jnp.float32),
            pltpu.VMEM((B1,), jnp.int32),
            pltpu.VMEM((B1,), jnp.int32),
            pltpu.VMEM((B1,), jnp.float32),
            pltpu.VMEM((B1,), jnp.float32),
            pltpu.VMEM((B1,), jnp.float32),
            pltpu.VMEM((B1, 16), jnp.float32),
            pltpu.VMEM_SHARED((NP, 16), jnp.float32),
        ],
        compiler_params=pltpu.CompilerParams(needs_layout_passes=False,
                                             use_tc_tiling_on_sc=False),
    )(pos_flat, snd, rcv, zeros16, ones)


def _p2a_body(zf_ref, tab_ref, s4_ref):
    zf = zf_ref[...]
    io = lax.broadcasted_iota(jnp.int32, (1, EMBED), 1).astype(jnp.float32)
    oh = (zf == io).astype(jnp.float32)
    s = jnp.dot(oh, tab_ref[...], precision=_HIGH)
    s4_ref[...] = jnp.stack([s[:, 32 * c:32 * c + 32] for c in range(4)], 0)


def _p2a_call(zf, tab_pad):
    return pl.pallas_call(
        _p2a_body,
        grid=(NP // 1024,),
        in_specs=[
            pl.BlockSpec((1024, 1), lambda i: (i, 0)),
            pl.BlockSpec((EMBED, EMBED), lambda i: (0, 0)),
        ],
        out_specs=pl.BlockSpec((4, 1024, 32), lambda i: (0, i, 0)),
        out_shape=jax.ShapeDtypeStruct((4, NP, 32), jnp.float32),
    )(zf, tab_pad)


def _p2b_body(rx_ref, ry_ref, rz_ref, W1r_ref, b1r_ref, W2_ref, b2r_ref,
              W3p_ref, b3p_ref, scalP_ref, ux_ref, uy_ref, uz_ref):
    rx, ry, rz = rx_ref[...], ry_ref[...], rz_ref[...]
    norm = jnp.sqrt(rx * rx + ry * ry + rz * rz)
    inv = 1.0 / jnp.maximum(norm, 1e-12)
    ux, uy, uz = rx * inv, ry * inv, rz * inv
    W1r, b1r = W1r_ref[...], b1r_ref[...]
    W2, b2r = W2_ref[...], b2r_ref[...]
    W3p, b3p = W3p_ref[...], b3p_ref[...]
    normT = norm.T
    slabs = []
    for r in range(8):
        nc = normT[:, r:r + 1]
        h = jax.nn.relu(nc * W1r + b1r)
        h = jax.nn.relu(jnp.dot(h, W2, precision=_HIGH) + b2r)
        scal = jnp.dot(h, W3p, precision=_HIGH) + b3p
        slabs.append(jnp.stack(
            [scal[:, 64 * c:64 * c + 64] for c in range(4)], 0))
    scalP_ref[...] = jnp.stack(slabs, axis=1)
    ux_ref[...], uy_ref[...], uz_ref[...] = ux, uy, uz


def _p2b_call(rx2, ry2, rz2, W1r, b1r, W2, b2r, W3p, b3p):
    eb = pl.BlockSpec((8, 128), lambda i: (i, 0))
    wf = lambda shape: pl.BlockSpec(shape, lambda i: tuple(0 for _ in shape))
    return pl.pallas_call(
        _p2b_body,
        grid=(EP // 1024,),
        in_specs=[eb, eb, eb,
                  wf((1, HID)), wf((1, HID)), wf((HID, HID)), wf((1, HID)),
                  wf((HID, 4 * HID)), wf((1, 4 * HID))],
        out_specs=[pl.BlockSpec((4, 8, 128, 64), lambda i: (0, i, 0, 0)),
                   eb, eb, eb],
        out_shape=[jax.ShapeDtypeStruct((4, EP // 128, 128, 64), jnp.float32)]
                  + [jax.ShapeDtypeStruct((EP // 128, 128), jnp.float32)] * 3,
    )(rx2, ry2, rz2, W1r, b1r, W2, b2r, W3p, b3p)


def _p3_body(scomb_hbm, scalP_hbm, ux_hbm, uy_hbm, uz_hbm, snd_hbm, rcv_hbm,
             zeros_hbm, zidx_hbm,
             acc_hbm,
             acc_sh,
             snd0_v, snd1_v, rcv0_v, rcv1_v, su0_v, su1_v,
             ux0_v, ux1_v, uy0_v, uy1_v, uz0_v, uz1_v,
             feat0_v, feat1_v, idx0_v, idx1_v, rsc0_v, rsc1_v,
             rows0_v, rows1_v,
             sm_snd0, sm_snd1, sm_rcv0, sm_rcv1, sm_su0, sm_su1,
             sm_ux0, sm_ux1, sm_uy0, sm_uy1, sm_uz0, sm_uz1,
             sm_ft0, sm_ft1, sm_sc0, sm_sc1):
    core = lax.axis_index("c")
    sub = lax.axis_index("s")
    rows_per_sub = NP // NS
    per_sub = EP // NS
    nbatch = per_sub // B3
    snd_b = (snd0_v, snd1_v)
    rcv_b = (rcv0_v, rcv1_v)
    su_b = (su0_v, su1_v)
    ux_b = (ux0_v, ux1_v)
    uy_b = (uy0_v, uy1_v)
    uz_b = (uz0_v, uz1_v)
    feat_b = (feat0_v, feat1_v)
    idx_b = (idx0_v, idx1_v)
    sm_snd = (sm_snd0, sm_snd1)
    sm_rcv = (sm_rcv0, sm_rcv1)
    sm_su = (sm_su0, sm_su1)
    sm_ux = (sm_ux0, sm_ux1)
    sm_uy = (sm_uy0, sm_uy1)
    sm_uz = (sm_uz0, sm_uz1)
    sm_ft = (sm_ft0, sm_ft1)
    rows_b = (rows0_v, rows1_v)
    rsc_b = (rsc0_v, rsc1_v)
    sm_sc = (sm_sc0, sm_sc1)
    ROWS_BYTES = B3 * 128 * 4

    def e_start(sub_, t):
        return jnp.minimum(sub_ * per_sub + t * B3, EP - B3)

    def issue_in(chunk, t, s):
        e0 = e_start(sub, t)
        pltpu.async_copy(snd_hbm.at[pl.ds(e0, B3)], snd_b[s], sm_snd[s])
        pltpu.async_copy(rcv_hbm.at[pl.ds(e0, B3)], rcv_b[s], sm_rcv[s])
        pltpu.async_copy(scalP_hbm.at[chunk, pl.ds(e0, B3)], su_b[s], sm_su[s])
        pltpu.async_copy(ux_hbm.at[pl.ds(e0, B3)], ux_b[s], sm_ux[s])
        pltpu.async_copy(uy_hbm.at[pl.ds(e0, B3)], uy_b[s], sm_uy[s])
        pltpu.async_copy(uz_hbm.at[pl.ds(e0, B3)], uz_b[s], sm_uz[s])

    def wait_in(chunk, t, s, which):
        e0 = e_start(sub, t)
        if which == "snd":
            pltpu.make_async_copy(snd_hbm.at[pl.ds(e0, B3)], snd_b[s],
                                  sm_snd[s]).wait()
        elif which == "rcv":
            pltpu.make_async_copy(rcv_hbm.at[pl.ds(e0, B3)], rcv_b[s],
                                  sm_rcv[s]).wait()
        elif which == "u":
            pltpu.make_async_copy(ux_hbm.at[pl.ds(e0, B3)], ux_b[s],
                                  sm_ux[s]).wait()
            pltpu.make_async_copy(uy_hbm.at[pl.ds(e0, B3)], uy_b[s],
                                  sm_uy[s]).wait()
            pltpu.make_async_copy(uz_hbm.at[pl.ds(e0, B3)], uz_b[s],
                                  sm_uz[s]).wait()
        else:
            pltpu.make_async_copy(scalP_hbm.at[chunk, pl.ds(e0, B3)],
                                  su_b[s], sm_su[s]).wait()

    def issue_feat(chunk, t, s):
        wait_in(chunk, t, s, "snd")
        base = chunk * NP
        for g in range(B3 // 16):
            sl = pl.ds(g * 16, 16)
            idx_b[s][sl] = snd_b[s][sl] + base
        pltpu.async_copy(scomb_hbm.at[idx_b[s]], feat_b[s], sm_ft[s])

    def wait_feat(s):
        pltpu.make_async_copy(scomb_hbm.at[idx_b[s]], feat_b[s],
                              sm_ft[s]).wait()

    def compute_rows(chunk, t, s):
        wait_feat(s)
        wait_in(chunk, t, s, "su")
        wait_in(chunk, t, s, "u")
        su = su_b[s]
        feat = feat_b[s]
        rows_v = rows_b[s]
        ux_v, uy_v, uz_v = ux_b[s], uy_b[s], uz_b[s]

        for ej in range(B3):
            f0 = feat[ej, pl.ds(0, 16)]
            f1 = feat[ej, pl.ds(16, 16)]
            wfs0 = f0 * su[ej, pl.ds(0, 16)]
            wfs1 = f1 * su[ej, pl.ds(16, 16)]
            wfv0 = f0 * su[ej, pl.ds(32, 16)]
            wfv1 = f1 * su[ej, pl.ds(48, 16)]
            eidx = jnp.full((16,), ej, jnp.int32)
            bux = plsc.load_gather(ux_v, [eidx])
            buy = plsc.load_gather(uy_v, [eidx])
            buz = plsc.load_gather(uz_v, [eidx])
            rows_v[ej, pl.ds(0, 16)] = wfs0
            rows_v[ej, pl.ds(16, 16)] = wfs1
            rows_v[ej, pl.ds(32, 16)] = wfv0 * bux
            rows_v[ej, pl.ds(48, 16)] = wfv1 * bux
            rows_v[ej, pl.ds(64, 16)] = wfv0 * buy
            rows_v[ej, pl.ds(80, 16)] = wfv1 * buy
            rows_v[ej, pl.ds(96, 16)] = wfv0 * buz
            rows_v[ej, pl.ds(112, 16)] = wfv1 * buz
    def do_scatter(chunk, t, s):
        wait_in(chunk, t, s, "rcv")
        pltpu.make_async_copy(rows_b[1 - s], acc_sh.at[rsc_b[1 - s]],
                              sm_sc0).wait()
        for g in range(B3 // 16):
            sl = pl.ds(g * 16, 16)
            rsc_b[s][sl] = rcv_b[s][sl]
        pltpu.async_copy(rows_b[s], acc_sh.at[rsc_b[s]], sm_sc0, add=True)

    def zero_acc():
        for j in range(rows_per_sub // 64):
            pltpu.sync_copy(zeros_hbm, acc_sh.at[pl.ds(sub * rows_per_sub + j * 64, 64)])

    def chunk_body(k, _):
        chunk = core * 2 + k
        zero_acc()
        plsc.subcore_barrier()

        pltpu.sync_copy(zeros_hbm, rows1_v.at[pl.ds(0, 64)])
        pltpu.sync_copy(zeros_hbm.at[pl.ds(0, 16)], rows1_v.at[pl.ds(64, 16)])
        pltpu.sync_copy(zidx_hbm, rsc1_v)
        pltpu.async_copy(rows1_v, acc_sh.at[rsc1_v], sm_sc0, add=True)

        issue_in(chunk, 0, 0)
        issue_feat(chunk, 0, 0)

        def pair(q, _):
            for p in range(2):
                t = 2 * q + p
                issue_in(chunk, t + 1, 1 - p)
                compute_rows(chunk, t, p)
                issue_feat(chunk, t + 1, 1 - p)
                do_scatter(chunk, t, p)
            return ()

        lax.fori_loop(0, nbatch // 2, pair, ())
        wait_in(chunk, nbatch, 0, "rcv")
        wait_in(chunk, nbatch, 0, "su")
        wait_in(chunk, nbatch, 0, "u")
        wait_feat(0)
        pltpu.make_async_copy(rows_b[1], acc_sh.at[rsc_b[1]], sm_sc0).wait()

        plsc.subcore_barrier()
        r0 = sub * rows_per_sub
        pltpu.sync_copy(acc_sh.at[pl.ds(r0, rows_per_sub)],
                        acc_hbm.at[chunk, pl.ds(r0, rows_per_sub)])
        plsc.subcore_barrier()
        return ()

    lax.fori_loop(0, 2, chunk_body, ())


def _p3_call(s_comb, scalP, ux, uy, uz, snd, rcv):
    zeros = jnp.zeros((64, 128), jnp.float32)
    zidx = jnp.zeros((B3,), jnp.int32)
    return pl.kernel(
        _p3_body,
        out_type=jax.ShapeDtypeStruct((4, NP, 128), jnp.float32),
        mesh=plsc.VectorSubcoreMesh(core_axis_name="c", subcore_axis_name="s"),
        scratch_types=[
            pltpu.VMEM_SHARED((NP, 128), jnp.float32),
            pltpu.VMEM((B3,), jnp.int32),
            pltpu.VMEM((B3,), jnp.int32),
            pltpu.VMEM((B3,), jnp.int32),
            pltpu.VMEM((B3,), jnp.int32),
            pltpu.VMEM((B3, 64), jnp.float32),
            pltpu.VMEM((B3, 64), jnp.float32),
            pltpu.VMEM((B3,), jnp.float32),
            pltpu.VMEM((B3,), jnp.float32),
            pltpu.VMEM((B3,), jnp.float32),
            pltpu.VMEM((B3,), jnp.float32),
            pltpu.VMEM((B3,), jnp.float32),
            pltpu.VMEM((B3,), jnp.float32),
            pltpu.VMEM((B3, 32), jnp.float32),
            pltpu.VMEM((B3, 32), jnp.float32),
            pltpu.VMEM((B3,), jnp.int32),
            pltpu.VMEM((B3,), jnp.int32),
            pltpu.VMEM((B3,), jnp.int32),
            pltpu.VMEM((B3,), jnp.int32),
            pltpu.VMEM((B3, 128), jnp.float32),
            pltpu.VMEM((B3, 128), jnp.float32),
        ] + [pltpu.SemaphoreType.DMA] * 16,
        compiler_params=pltpu.CompilerParams(needs_layout_passes=False,
                                             use_tc_tiling_on_sc=False),
    )(s_comb, scalP, ux, uy, uz, snd, rcv, zeros, zidx)


def _p4_body(acc_ref, cnt_ref, s4_ref, Wgs_ref, Wgv_ref, Wos_ref, Wov_ref,
             Wrs_ref, Wrv_ref, out_ref):
    i = pl.program_id(0)
    cnt = jnp.maximum(cnt_ref[0, :, 0:1] + cnt_ref[1, :, 0:1], 1.0)
    inv = 1.0 / cnt
    acc = acc_ref[...]
    agg_s = jnp.concatenate([acc[c, :, 0:32] for c in range(4)], 1) * inv
    agg_vx = jnp.concatenate([acc[c, :, 32:64] for c in range(4)], 1) * inv
    agg_vy = jnp.concatenate([acc[c, :, 64:96] for c in range(4)], 1) * inv
    agg_vz = jnp.concatenate([acc[c, :, 96:128] for c in range(4)], 1) * inv
    exp_s = jnp.dot(agg_s, Wgs_ref[...], precision=_HIGH)
    act_s = jax.nn.gelu(exp_s[:, :2 * EMBED])
    gates = jax.nn.sigmoid(exp_s[:, 2 * EMBED:])
    Wgv = Wgv_ref[...]
    gvx = jnp.dot(agg_vx, Wgv, precision=_HIGH) * gates
    gvy = jnp.dot(agg_vy, Wgv, precision=_HIGH) * gates
    gvz = jnp.dot(agg_vz, Wgv, precision=_HIGH) * gates
    skip = jnp.concatenate([s4_ref[c] for c in range(4)], 1)
    cat = jnp.concatenate([act_s, skip], 1)
    out_s = jnp.dot(cat, Wos_ref[...], precision=_HIGH)
    Wov = Wov_ref[...]
    ovx = jnp.dot(gvx, Wov, precision=_HIGH)
    ovy = jnp.dot(gvy, Wov, precision=_HIGH)
    ovz = jnp.dot(gvz, Wov, precision=_HIGH)
    inv_v = ovx * ovx + ovy * ovy + ovz * ovz
    node = (jnp.dot(out_s, Wrs_ref[...], precision=_HIGH)
            + jnp.dot(inv_v, Wrv_ref[...], precision=_HIGH))
    rowid = i * 1024 + lax.broadcasted_iota(jnp.int32, (1024, 1), 0)
    node = jnp.where(rowid < N, node, 0.0)
    psum = jnp.sum(node, keepdims=True).reshape(1, 1)

    @pl.when(i == 0)
    def _():
        out_ref[...] = jnp.zeros((1, 1), jnp.float32)
    out_ref[...] += psum


def _p4_call(acc, cnt, s4, Wg_s, Wg_v, Wo_s, Wo_v, W_read_s, W_read_v):
    wf = lambda shape: pl.BlockSpec(shape, lambda i: tuple(0 for _ in shape))
    return pl.pallas_call(
        _p4_body,
        grid=(NP // 1024,),
        in_specs=[
            pl.BlockSpec((4, 1024, 128), lambda i: (0, i, 0)),
            pl.BlockSpec((2, 1024, 16), lambda i: (0, i, 0)),
            pl.BlockSpec((4, 1024, 32), lambda i: (0, i, 0)),
            wf((EMBED, 3 * EMBED)), wf((EMBED, EMBED)),
            wf((3 * EMBED, EMBED)), wf((EMBED, VEC_OUT)),
            wf((EMBED, 1)), wf((VEC_OUT, 1)),
        ],
        out_specs=pl.BlockSpec((1, 1), lambda i: (0, 0)),
        out_shape=jax.ShapeDtypeStruct((1, 1), jnp.float32),
    )(acc, cnt, s4, Wg_s, Wg_v, Wo_s, Wo_v, W_read_s, W_read_v)


@jax.jit
def kernel(atomic_numbers, positions, senders, receivers, embed_table,
           W1, b1, W2, b2, W3, b3, Wg_s, Wg_v, Wo_s, Wo_v,
           W_read_s, W_read_v):
    snd_p = jnp.concatenate([senders, jnp.zeros((EP - E,), jnp.int32)])
    rcv_p = jnp.concatenate([receivers, jnp.full((EP - E,), N, jnp.int32)])
    pos_flat = positions.reshape(-1)
    zf = jnp.concatenate([atomic_numbers.astype(jnp.float32),
                          jnp.zeros((NP - N,), jnp.float32)]).reshape(NP, 1)
    tab_pad = jnp.concatenate(
        [embed_table, jnp.zeros((EMBED - embed_table.shape[0], EMBED),
                                jnp.float32)], 0)

    s4 = _p2a_call(zf, tab_pad)
    s_comb = s4.reshape(4 * NP, 32)

    relx, rely, relz, cnt = _p1_call(pos_flat, snd_p, rcv_p)

    W3s, W3v = W3[:, :EMBED], W3[:, EMBED:]
    W3p = jnp.concatenate(
        [jnp.concatenate([W3s[:, 32 * c:32 * c + 32],
                          W3v[:, 32 * c:32 * c + 32]], 1) for c in range(4)], 1)
    b3s, b3v = b3[:EMBED], b3[EMBED:]
    b3p = jnp.concatenate(
        [jnp.concatenate([b3s[32 * c:32 * c + 32],
                          b3v[32 * c:32 * c + 32]]) for c in range(4)])
    scalP, ux2, uy2, uz2 = _p2b_call(
        relx.reshape(EP // 128, 128), rely.reshape(EP // 128, 128),
        relz.reshape(EP // 128, 128),
        W1, b1.reshape(1, HID), W2, b2.reshape(1, HID),
        W3p, b3p.reshape(1, 4 * HID))

    acc = _p3_call(s_comb, scalP.reshape(4, EP, 64), ux2.reshape(EP),
                   uy2.reshape(EP), uz2.reshape(EP), snd_p, rcv_p)

    total = _p4_call(acc, cnt, s4, Wg_s, Wg_v, Wo_s, Wo_v,
                     W_read_s, W_read_v)
    return total[0, 0] / N


# speedup vs baseline: 1.1341x; 1.1341x over previous
import functools
import jax
import jax.numpy as jnp
from jax import lax
from jax.experimental import pallas as pl
from jax.experimental.pallas import tpu as pltpu
from jax.experimental.pallas import tpu_sc as plsc

N = 10000
E = 160000
EMBED = 128
HID = 64
VEC_OUT = 64

NC = 2
NS = 16
EP = 163840
NP = 10240
B3 = 80
B1 = 512

_HIGH = jax.lax.Precision.HIGHEST


def _p1_body(pos_hbm, snd_hbm, rcv_hbm, zeros16_hbm, ones_hbm,
             relx_hbm, rely_hbm, relz_hbm, cnt_hbm,
             pos_v, snd_v, rcv_v, ox_v, oy_v, oz_v, ones_v, cnt_sh):
    core = lax.axis_index("c")
    sub = lax.axis_index("s")
    wid = sub * NC + core
    rows_per_sub = NP // NS
    pltpu.sync_copy(pos_hbm, pos_v)
    pltpu.sync_copy(ones_hbm, ones_v)
    for j in range(rows_per_sub // 64):
        pltpu.sync_copy(zeros16_hbm, cnt_sh.at[pl.ds(sub * rows_per_sub + j * 64, 64)])
    plsc.subcore_barrier()
    per_tile = EP // (NC * NS)

    def batch(b, _):
        e0 = wid * per_tile + b * B1
        pltpu.sync_copy(snd_hbm.at[pl.ds(e0, B1)], snd_v)
        pltpu.sync_copy(rcv_hbm.at[pl.ds(e0, B1)], rcv_v)
        for g in range(B1 // 16):
            sl = pl.ds(g * 16, 16)
            s3 = snd_v[sl] * 3
            r3 = rcv_v[sl] * 3
            for d, ref in ((0, ox_v), (1, oy_v), (2, oz_v)):
                ps = plsc.load_gather(pos_v, [s3 + d])
                pr = plsc.load_gather(pos_v, [r3 + d])
                ref[sl] = pr - ps
        pltpu.sync_copy(ox_v, relx_hbm.at[pl.ds(e0, B1)])
        pltpu.sync_copy(oy_v, rely_hbm.at[pl.ds(e0, B1)])
        pltpu.sync_copy(oz_v, relz_hbm.at[pl.ds(e0, B1)])
        pltpu.sync_copy(ones_v, cnt_sh.at[rcv_v], add=True)
        return ()

    lax.fori_loop(0, per_tile // B1, batch, ())
    plsc.subcore_barrier()
    r0 = sub * rows_per_sub
    pltpu.sync_copy(cnt_sh.at[pl.ds(r0, rows_per_sub)],
                    cnt_hbm.at[core, pl.ds(r0, rows_per_sub)])


def _p1_call(pos_flat, snd, rcv):
    zeros16 = jnp.zeros((64, 16), jnp.float32)
    ones = jnp.ones((B1, 16), jnp.float32)
    return pl.kernel(
        _p1_body,
        out_type=(jax.ShapeDtypeStruct((EP,), jnp.float32),) * 3
                 + (jax.ShapeDtypeStruct((2, NP, 16), jnp.float32),),
        mesh=plsc.VectorSubcoreMesh(core_axis_name="c", subcore_axis_name="s"),
        scratch_types=[
            pltpu.VMEM((N * 3,), jnp.float32),
            pltpu.VMEM((B1,), jnp.int32),
            pltpu.VMEM((B1,), jnp.int32),
            pltpu.VMEM((B1,), jnp.float32),
            pltpu.VMEM((B1,), jnp.float32),
            pltpu.VMEM((B1,), jnp.float32),
            pltpu.VMEM((B1, 16), jnp.float32),
            pltpu.VMEM_SHARED((NP, 16), jnp.float32),
        ],
        compiler_params=pltpu.CompilerParams(needs_layout_passes=False,
                                             use_tc_tiling_on_sc=False),
    )(pos_flat, snd, rcv, zeros16, ones)


def _p2a_body(zf_ref, tab_ref, s4_ref):
    zf = zf_ref[...]
    io = lax.broadcasted_iota(jnp.int32, (1, EMBED), 1).astype(jnp.float32)
    oh = (zf == io).astype(jnp.float32)
    s = jnp.dot(oh, tab_ref[...], precision=_HIGH)
    s4_ref[...] = jnp.stack([s[:, 32 * c:32 * c + 32] for c in range(4)], 0)


def _p2a_call(zf, tab_pad):
    return pl.pallas_call(
        _p2a_body,
        grid=(NP // 1024,),
        in_specs=[
            pl.BlockSpec((1024, 1), lambda i: (i, 0)),
            pl.BlockSpec((EMBED, EMBED), lambda i: (0, 0)),
        ],
        out_specs=pl.BlockSpec((4, 1024, 32), lambda i: (0, i, 0)),
        out_shape=jax.ShapeDtypeStruct((4, NP, 32), jnp.float32),
    )(zf, tab_pad)


def _p2b_body(rx_ref, ry_ref, rz_ref, W1r_ref, b1r_ref, W2_ref, b2r_ref,
              W3p_ref, b3p_ref, scalP_ref, ux_ref, uy_ref, uz_ref):
    rx, ry, rz = rx_ref[...], ry_ref[...], rz_ref[...]
    norm = jnp.sqrt(rx * rx + ry * ry + rz * rz)
    inv = 1.0 / jnp.maximum(norm, 1e-12)
    ux, uy, uz = rx * inv, ry * inv, rz * inv
    W1r, b1r = W1r_ref[...], b1r_ref[...]
    W2, b2r = W2_ref[...], b2r_ref[...]
    W3p, b3p = W3p_ref[...], b3p_ref[...]
    normT = norm.T
    slabs = []
    for r in range(8):
        nc = normT[:, r:r + 1]
        h = jax.nn.relu(nc * W1r + b1r)
        h = jax.nn.relu(jnp.dot(h, W2, precision=_HIGH) + b2r)
        scal = jnp.dot(h, W3p, precision=_HIGH) + b3p
        slabs.append(jnp.stack(
            [scal[:, 64 * c:64 * c + 64] for c in range(4)], 0))
    scalP_ref[...] = jnp.stack(slabs, axis=1)
    ux_ref[...], uy_ref[...], uz_ref[...] = ux, uy, uz


def _p2b_call(rx2, ry2, rz2, W1r, b1r, W2, b2r, W3p, b3p):
    eb = pl.BlockSpec((8, 128), lambda i: (i, 0))
    wf = lambda shape: pl.BlockSpec(shape, lambda i: tuple(0 for _ in shape))
    return pl.pallas_call(
        _p2b_body,
        grid=(EP // 1024,),
        in_specs=[eb, eb, eb,
                  wf((1, HID)), wf((1, HID)), wf((HID, HID)), wf((1, HID)),
                  wf((HID, 4 * HID)), wf((1, 4 * HID))],
        out_specs=[pl.BlockSpec((4, 8, 128, 64), lambda i: (0, i, 0, 0)),
                   eb, eb, eb],
        out_shape=[jax.ShapeDtypeStruct((4, EP // 128, 128, 64), jnp.float32)]
                  + [jax.ShapeDtypeStruct((EP // 128, 128), jnp.float32)] * 3,
    )(rx2, ry2, rz2, W1r, b1r, W2, b2r, W3p, b3p)


def _p3_body(scomb_hbm, scalP_hbm, ux_hbm, uy_hbm, uz_hbm, snd_hbm, rcv_hbm,
             zeros_hbm, zidx_hbm,
             acc_hbm,
             acc_sh,
             snd0_v, snd1_v, rcv0_v, rcv1_v, su0_v, su1_v,
             ux0_v, ux1_v, uy0_v, uy1_v, uz0_v, uz1_v,
             feat0_v, feat1_v, idx0_v, idx1_v, rsc0_v, rsc1_v,
             rows0_v, rows1_v,
             sm_snd0, sm_snd1, sm_rcv0, sm_rcv1, sm_su0, sm_su1,
             sm_ux0, sm_ux1, sm_uy0, sm_uy1, sm_uz0, sm_uz1,
             sm_ft0, sm_ft1, sm_sc0, sm_sc1):
    core = lax.axis_index("c")
    sub = lax.axis_index("s")
    rows_per_sub = NP // NS
    per_sub = EP // NS
    nbatch = per_sub // B3
    snd_b = (snd0_v, snd1_v)
    rcv_b = (rcv0_v, rcv1_v)
    su_b = (su0_v, su1_v)
    ux_b = (ux0_v, ux1_v)
    uy_b = (uy0_v, uy1_v)
    uz_b = (uz0_v, uz1_v)
    feat_b = (feat0_v, feat1_v)
    idx_b = (idx0_v, idx1_v)
    sm_snd = (sm_snd0, sm_snd1)
    sm_rcv = (sm_rcv0, sm_rcv1)
    sm_su = (sm_su0, sm_su1)
    sm_ux = (sm_ux0, sm_ux1)
    sm_uy = (sm_uy0, sm_uy1)
    sm_uz = (sm_uz0, sm_uz1)
    sm_ft = (sm_ft0, sm_ft1)
    rows_b = (rows0_v, rows1_v)
    rsc_b = (rsc0_v, rsc1_v)
    sm_sc = (sm_sc0, sm_sc1)
    ROWS_BYTES = B3 * 128 * 4

    def e_start(sub_, t):
        return jnp.minimum(sub_ * per_sub + t * B3, EP - B3)

    def issue_in(chunk, t, s):
        e0 = e_start(sub, t)
        pltpu.async_copy(snd_hbm.at[pl.ds(e0, B3)], snd_b[s], sm_snd[s])
        pltpu.async_copy(rcv_hbm.at[pl.ds(e0, B3)], rcv_b[s], sm_rcv[s])
        pltpu.async_copy(scalP_hbm.at[chunk, pl.ds(e0, B3)], su_b[s], sm_su[s])
        pltpu.async_copy(ux_hbm.at[pl.ds(e0, B3)], ux_b[s], sm_ux[s])
        pltpu.async_copy(uy_hbm.at[pl.ds(e0, B3)], uy_b[s], sm_uy[s])
        pltpu.async_copy(uz_hbm.at[pl.ds(e0, B3)], uz_b[s], sm_uz[s])

    def wait_in(chunk, t, s, which):
        e0 = e_start(sub, t)
        if which == "snd":
            pltpu.make_async_copy(snd_hbm.at[pl.ds(e0, B3)], snd_b[s],
                                  sm_snd[s]).wait()
        elif which == "rcv":
            pltpu.make_async_copy(rcv_hbm.at[pl.ds(e0, B3)], rcv_b[s],
                                  sm_rcv[s]).wait()
        elif which == "u":
            pltpu.make_async_copy(ux_hbm.at[pl.ds(e0, B3)], ux_b[s],
                                  sm_ux[s]).wait()
            pltpu.make_async_copy(uy_hbm.at[pl.ds(e0, B3)], uy_b[s],
                                  sm_uy[s]).wait()
            pltpu.make_async_copy(uz_hbm.at[pl.ds(e0, B3)], uz_b[s],
                                  sm_uz[s]).wait()
        else:
            pltpu.make_async_copy(scalP_hbm.at[chunk, pl.ds(e0, B3)],
                                  su_b[s], sm_su[s]).wait()

    def issue_feat(chunk, t, s):
        wait_in(chunk, t, s, "snd")
        base = chunk * NP
        for g in range(B3 // 16):
            sl = pl.ds(g * 16, 16)
            idx_b[s][sl] = snd_b[s][sl] + base
        pltpu.async_copy(scomb_hbm.at[idx_b[s]], feat_b[s], sm_ft[s])

    def wait_feat(s):
        pltpu.make_async_copy(scomb_hbm.at[idx_b[s]], feat_b[s],
                              sm_ft[s]).wait()

    def compute_rows(chunk, t, s):
        wait_feat(s)
        wait_in(chunk, t, s, "su")
        wait_in(chunk, t, s, "u")
        su = su_b[s]
        feat = feat_b[s]
        rows_v = rows_b[s]
        ux_v, uy_v, uz_v = ux_b[s], uy_b[s], uz_b[s]

        @plsc.parallel_loop(0, B3, unroll=8)
        def _(ej):
            f0 = feat[ej, pl.ds(0, 16)]
            f1 = feat[ej, pl.ds(16, 16)]
            wfs0 = f0 * su[ej, pl.ds(0, 16)]
            wfs1 = f1 * su[ej, pl.ds(16, 16)]
            wfv0 = f0 * su[ej, pl.ds(32, 16)]
            wfv1 = f1 * su[ej, pl.ds(48, 16)]
            eidx = jnp.full((16,), ej, jnp.int32)
            bux = plsc.load_gather(ux_v, [eidx])
            buy = plsc.load_gather(uy_v, [eidx])
            buz = plsc.load_gather(uz_v, [eidx])
            rows_v[ej, pl.ds(0, 16)] = wfs0
            rows_v[ej, pl.ds(16, 16)] = wfs1
            rows_v[ej, pl.ds(32, 16)] = wfv0 * bux
            rows_v[ej, pl.ds(48, 16)] = wfv1 * bux
            rows_v[ej, pl.ds(64, 16)] = wfv0 * buy
            rows_v[ej, pl.ds(80, 16)] = wfv1 * buy
            rows_v[ej, pl.ds(96, 16)] = wfv0 * buz
            rows_v[ej, pl.ds(112, 16)] = wfv1 * buz
    def do_scatter(chunk, t, s):
        wait_in(chunk, t, s, "rcv")
        pltpu.make_async_copy(rows_b[1 - s], acc_sh.at[rsc_b[1 - s]],
                              sm_sc0).wait()
        for g in range(B3 // 16):
            sl = pl.ds(g * 16, 16)
            rsc_b[s][sl] = rcv_b[s][sl]
        pltpu.async_copy(rows_b[s], acc_sh.at[rsc_b[s]], sm_sc0, add=True)

    def zero_acc():
        for j in range(rows_per_sub // 64):
            pltpu.sync_copy(zeros_hbm, acc_sh.at[pl.ds(sub * rows_per_sub + j * 64, 64)])

    def chunk_body(k, _):
        chunk = core * 2 + k
        zero_acc()
        plsc.subcore_barrier()

        pltpu.sync_copy(zeros_hbm, rows1_v.at[pl.ds(0, 64)])
        pltpu.sync_copy(zeros_hbm.at[pl.ds(0, 16)], rows1_v.at[pl.ds(64, 16)])
        pltpu.sync_copy(zidx_hbm, rsc1_v)
        pltpu.async_copy(rows1_v, acc_sh.at[rsc1_v], sm_sc0, add=True)

        issue_in(chunk, 0, 0)
        issue_feat(chunk, 0, 0)

        def pair(q, _):
            for p in range(2):
                t = 2 * q + p
                issue_in(chunk, t + 1, 1 - p)
                compute_rows(chunk, t, p)
                issue_feat(chunk, t + 1, 1 - p)
                do_scatter(chunk, t, p)
            return ()

        lax.fori_loop(0, nbatch // 2, pair, ())
        wait_in(chunk, nbatch, 0, "rcv")
        wait_in(chunk, nbatch, 0, "su")
        wait_in(chunk, nbatch, 0, "u")
        wait_feat(0)
        pltpu.make_async_copy(rows_b[1], acc_sh.at[rsc_b[1]], sm_sc0).wait()

        plsc.subcore_barrier()
        r0 = sub * rows_per_sub
        pltpu.sync_copy(acc_sh.at[pl.ds(r0, rows_per_sub)],
                        acc_hbm.at[chunk, pl.ds(r0, rows_per_sub)])
        plsc.subcore_barrier()
        return ()

    lax.fori_loop(0, 2, chunk_body, ())


def _p3_call(s_comb, scalP, ux, uy, uz, snd, rcv):
    zeros = jnp.zeros((64, 128), jnp.float32)
    zidx = jnp.zeros((B3,), jnp.int32)
    return pl.kernel(
        _p3_body,
        out_type=jax.ShapeDtypeStruct((4, NP, 128), jnp.float32),
        mesh=plsc.VectorSubcoreMesh(core_axis_name="c", subcore_axis_name="s"),
        scratch_types=[
            pltpu.VMEM_SHARED((NP, 128), jnp.float32),
            pltpu.VMEM((B3,), jnp.int32),
            pltpu.VMEM((B3,), jnp.int32),
            pltpu.VMEM((B3,), jnp.int32),
            pltpu.VMEM((B3,), jnp.int32),
            pltpu.VMEM((B3, 64), jnp.float32),
            pltpu.VMEM((B3, 64), jnp.float32),
            pltpu.VMEM((B3,), jnp.float32),
            pltpu.VMEM((B3,), jnp.float32),
            pltpu.VMEM((B3,), jnp.float32),
            pltpu.VMEM((B3,), jnp.float32),
            pltpu.VMEM((B3,), jnp.float32),
            pltpu.VMEM((B3,), jnp.float32),
            pltpu.VMEM((B3, 32), jnp.float32),
            pltpu.VMEM((B3, 32), jnp.float32),
            pltpu.VMEM((B3,), jnp.int32),
            pltpu.VMEM((B3,), jnp.int32),
            pltpu.VMEM((B3,), jnp.int32),
            pltpu.VMEM((B3,), jnp.int32),
            pltpu.VMEM((B3, 128), jnp.float32),
            pltpu.VMEM((B3, 128), jnp.float32),
        ] + [pltpu.SemaphoreType.DMA] * 16,
        compiler_params=pltpu.CompilerParams(needs_layout_passes=False,
                                             use_tc_tiling_on_sc=False),
    )(s_comb, scalP, ux, uy, uz, snd, rcv, zeros, zidx)


def _p4_body(acc_ref, cnt_ref, s4_ref, Wgs_ref, Wgv_ref, Wos_ref, Wov_ref,
             Wrs_ref, Wrv_ref, out_ref):
    i = pl.program_id(0)
    cnt = jnp.maximum(cnt_ref[0, :, 0:1] + cnt_ref[1, :, 0:1], 1.0)
    inv = 1.0 / cnt
    acc = acc_ref[...]
    agg_s = jnp.concatenate([acc[c, :, 0:32] for c in range(4)], 1) * inv
    agg_vx = jnp.concatenate([acc[c, :, 32:64] for c in range(4)], 1) * inv
    agg_vy = jnp.concatenate([acc[c, :, 64:96] for c in range(4)], 1) * inv
    agg_vz = jnp.concatenate([acc[c, :, 96:128] for c in range(4)], 1) * inv
    exp_s = jnp.dot(agg_s, Wgs_ref[...], precision=_HIGH)
    act_s = jax.nn.gelu(exp_s[:, :2 * EMBED])
    gates = jax.nn.sigmoid(exp_s[:, 2 * EMBED:])
    Wgv = Wgv_ref[...]
    gvx = jnp.dot(agg_vx, Wgv, precision=_HIGH) * gates
    gvy = jnp.dot(agg_vy, Wgv, precision=_HIGH) * gates
    gvz = jnp.dot(agg_vz, Wgv, precision=_HIGH) * gates
    skip = jnp.concatenate([s4_ref[c] for c in range(4)], 1)
    cat = jnp.concatenate([act_s, skip], 1)
    out_s = jnp.dot(cat, Wos_ref[...], precision=_HIGH)
    Wov = Wov_ref[...]
    ovx = jnp.dot(gvx, Wov, precision=_HIGH)
    ovy = jnp.dot(gvy, Wov, precision=_HIGH)
    ovz = jnp.dot(gvz, Wov, precision=_HIGH)
    inv_v = ovx * ovx + ovy * ovy + ovz * ovz
    node = (jnp.dot(out_s, Wrs_ref[...], precision=_HIGH)
            + jnp.dot(inv_v, Wrv_ref[...], precision=_HIGH))
    rowid = i * 1024 + lax.broadcasted_iota(jnp.int32, (1024, 1), 0)
    node = jnp.where(rowid < N, node, 0.0)
    psum = jnp.sum(node, keepdims=True).reshape(1, 1)

    @pl.when(i == 0)
    def _():
        out_ref[...] = jnp.zeros((1, 1), jnp.float32)
    out_ref[...] += psum


def _p4_call(acc, cnt, s4, Wg_s, Wg_v, Wo_s, Wo_v, W_read_s, W_read_v):
    wf = lambda shape: pl.BlockSpec(shape, lambda i: tuple(0 for _ in shape))
    return pl.pallas_call(
        _p4_body,
        grid=(NP // 1024,),
        in_specs=[
            pl.BlockSpec((4, 1024, 128), lambda i: (0, i, 0)),
            pl.BlockSpec((2, 1024, 16), lambda i: (0, i, 0)),
            pl.BlockSpec((4, 1024, 32), lambda i: (0, i, 0)),
            wf((EMBED, 3 * EMBED)), wf((EMBED, EMBED)),
            wf((3 * EMBED, EMBED)), wf((EMBED, VEC_OUT)),
            wf((EMBED, 1)), wf((VEC_OUT, 1)),
        ],
        out_specs=pl.BlockSpec((1, 1), lambda i: (0, 0)),
        out_shape=jax.ShapeDtypeStruct((1, 1), jnp.float32),
    )(acc, cnt, s4, Wg_s, Wg_v, Wo_s, Wo_v, W_read_s, W_read_v)


@jax.jit
def kernel(atomic_numbers, positions, senders, receivers, embed_table,
           W1, b1, W2, b2, W3, b3, Wg_s, Wg_v, Wo_s, Wo_v,
           W_read_s, W_read_v):
    snd_p = jnp.concatenate([senders, jnp.zeros((EP - E,), jnp.int32)])
    rcv_p = jnp.concatenate([receivers, jnp.full((EP - E,), N, jnp.int32)])
    pos_flat = positions.reshape(-1)
    zf = jnp.concatenate([atomic_numbers.astype(jnp.float32),
                          jnp.zeros((NP - N,), jnp.float32)]).reshape(NP, 1)
    tab_pad = jnp.concatenate(
        [embed_table, jnp.zeros((EMBED - embed_table.shape[0], EMBED),
                                jnp.float32)], 0)

    s4 = _p2a_call(zf, tab_pad)
    s_comb = s4.reshape(4 * NP, 32)

    relx, rely, relz, cnt = _p1_call(pos_flat, snd_p, rcv_p)

    W3s, W3v = W3[:, :EMBED], W3[:, EMBED:]
    W3p = jnp.concatenate(
        [jnp.concatenate([W3s[:, 32 * c:32 * c + 32],
                          W3v[:, 32 * c:32 * c + 32]], 1) for c in range(4)], 1)
    b3s, b3v = b3[:EMBED], b3[EMBED:]
    b3p = jnp.concatenate(
        [jnp.concatenate([b3s[32 * c:32 * c + 32],
                          b3v[32 * c:32 * c + 32]]) for c in range(4)])
    scalP, ux2, uy2, uz2 = _p2b_call(
        relx.reshape(EP // 128, 128), rely.reshape(EP // 128, 128),
        relz.reshape(EP // 128, 128),
        W1, b1.reshape(1, HID), W2, b2.reshape(1, HID),
        W3p, b3p.reshape(1, 4 * HID))

    acc = _p3_call(s_comb, scalP.reshape(4, EP, 64), ux2.reshape(EP),
                   uy2.reshape(EP), uz2.reshape(EP), snd_p, rcv_p)

    total = _p4_call(acc, cnt, s4, Wg_s, Wg_v, Wo_s, Wo_v,
                     W_read_s, W_read_v)
    return total[0, 0] / N


# speedup vs baseline: 1.1393x; 1.0046x over previous
import functools
import jax
import jax.numpy as jnp
from jax import lax
from jax.experimental import pallas as pl
from jax.experimental.pallas import tpu as pltpu
from jax.experimental.pallas import tpu_sc as plsc

N = 10000
E = 160000
EMBED = 128
HID = 64
VEC_OUT = 64

NC = 2
NS = 16
EP = 163840
NP = 10240
B3 = 80
B1 = 512

_HIGH = jax.lax.Precision.HIGHEST


def _p1_body(pos_hbm, snd_hbm, rcv_hbm, zeros16_hbm, ones_hbm,
             relx_hbm, rely_hbm, relz_hbm, cnt_hbm,
             pos_v, snd_v, rcv_v, ox_v, oy_v, oz_v, ones_v, cnt_sh):
    core = lax.axis_index("c")
    sub = lax.axis_index("s")
    wid = sub * NC + core
    rows_per_sub = NP // NS
    pltpu.sync_copy(pos_hbm, pos_v)
    pltpu.sync_copy(ones_hbm, ones_v)
    for j in range(rows_per_sub // 64):
        pltpu.sync_copy(zeros16_hbm, cnt_sh.at[pl.ds(sub * rows_per_sub + j * 64, 64)])
    plsc.subcore_barrier()
    per_tile = EP // (NC * NS)

    def batch(b, _):
        e0 = wid * per_tile + b * B1
        pltpu.sync_copy(snd_hbm.at[pl.ds(e0, B1)], snd_v)
        pltpu.sync_copy(rcv_hbm.at[pl.ds(e0, B1)], rcv_v)
        for g in range(B1 // 16):
            sl = pl.ds(g * 16, 16)
            s3 = snd_v[sl] * 3
            r3 = rcv_v[sl] * 3
            for d, ref in ((0, ox_v), (1, oy_v), (2, oz_v)):
                ps = plsc.load_gather(pos_v, [s3 + d])
                pr = plsc.load_gather(pos_v, [r3 + d])
                ref[sl] = pr - ps
        pltpu.sync_copy(ox_v, relx_hbm.at[pl.ds(e0, B1)])
        pltpu.sync_copy(oy_v, rely_hbm.at[pl.ds(e0, B1)])
        pltpu.sync_copy(oz_v, relz_hbm.at[pl.ds(e0, B1)])
        pltpu.sync_copy(ones_v, cnt_sh.at[rcv_v], add=True)
        return ()

    lax.fori_loop(0, per_tile // B1, batch, ())
    plsc.subcore_barrier()
    r0 = sub * rows_per_sub
    pltpu.sync_copy(cnt_sh.at[pl.ds(r0, rows_per_sub)],
                    cnt_hbm.at[core, pl.ds(r0, rows_per_sub)])


def _p1_call(pos_flat, snd, rcv):
    zeros16 = jnp.zeros((64, 16), jnp.float32)
    ones = jnp.ones((B1, 16), jnp.float32)
    return pl.kernel(
        _p1_body,
        out_type=(jax.ShapeDtypeStruct((EP,), jnp.float32),) * 3
                 + (jax.ShapeDtypeStruct((2, NP, 16), jnp.float32),),
        mesh=plsc.VectorSubcoreMesh(core_axis_name="c", subcore_axis_name="s"),
        scratch_types=[
            pltpu.VMEM((N * 3,), jnp.float32),
            pltpu.VMEM((B1,), jnp.int32),
            pltpu.VMEM((B1,), jnp.int32),
            pltpu.VMEM((B1,), jnp.float32),
            pltpu.VMEM((B1,), jnp.float32),
            pltpu.VMEM((B1,), jnp.float32),
            pltpu.VMEM((B1, 16), jnp.float32),
            pltpu.VMEM_SHARED((NP, 16), jnp.float32),
        ],
        compiler_params=pltpu.CompilerParams(needs_layout_passes=False,
                                             use_tc_tiling_on_sc=False),
    )(pos_flat, snd, rcv, zeros16, ones)


def _p2a_body(zf_ref, tab_ref, s4_ref):
    zf = zf_ref[...]
    io = lax.broadcasted_iota(jnp.int32, (1, EMBED), 1).astype(jnp.float32)
    oh = (zf == io).astype(jnp.float32)
    s = jnp.dot(oh, tab_ref[...], precision=_HIGH)
    s4_ref[...] = jnp.stack([s[:, 32 * c:32 * c + 32] for c in range(4)], 0)


def _p2a_call(zf, tab_pad):
    return pl.pallas_call(
        _p2a_body,
        grid=(NP // 1024,),
        in_specs=[
            pl.BlockSpec((1024, 1), lambda i: (i, 0)),
            pl.BlockSpec((EMBED, EMBED), lambda i: (0, 0)),
        ],
        out_specs=pl.BlockSpec((4, 1024, 32), lambda i: (0, i, 0)),
        out_shape=jax.ShapeDtypeStruct((4, NP, 32), jnp.float32),
    )(zf, tab_pad)


def _p2b_body(rx_ref, ry_ref, rz_ref, W1r_ref, b1r_ref, W2_ref, b2r_ref,
              W3p_ref, b3p_ref, scalP_ref, ux_ref, uy_ref, uz_ref):
    rx, ry, rz = rx_ref[...], ry_ref[...], rz_ref[...]
    norm = jnp.sqrt(rx * rx + ry * ry + rz * rz)
    inv = 1.0 / jnp.maximum(norm, 1e-12)
    ux, uy, uz = rx * inv, ry * inv, rz * inv
    W1r, b1r = W1r_ref[...], b1r_ref[...]
    W2, b2r = W2_ref[...], b2r_ref[...]
    W3p, b3p = W3p_ref[...], b3p_ref[...]
    normT = norm.T
    slabs = []
    for r in range(8):
        nc = normT[:, r:r + 1]
        h = jax.nn.relu(nc * W1r + b1r)
        h = jax.nn.relu(jnp.dot(h, W2, precision=_HIGH) + b2r)
        scal = jnp.dot(h, W3p, precision=_HIGH) + b3p
        slabs.append(jnp.stack(
            [scal[:, 64 * c:64 * c + 64] for c in range(4)], 0))
    scalP_ref[...] = jnp.stack(slabs, axis=1)
    ux_ref[...], uy_ref[...], uz_ref[...] = ux, uy, uz


def _p2b_call(rx2, ry2, rz2, W1r, b1r, W2, b2r, W3p, b3p):
    eb = pl.BlockSpec((8, 128), lambda i: (i, 0))
    wf = lambda shape: pl.BlockSpec(shape, lambda i: tuple(0 for _ in shape))
    return pl.pallas_call(
        _p2b_body,
        grid=(EP // 1024,),
        in_specs=[eb, eb, eb,
                  wf((1, HID)), wf((1, HID)), wf((HID, HID)), wf((1, HID)),
                  wf((HID, 4 * HID)), wf((1, 4 * HID))],
        out_specs=[pl.BlockSpec((4, 8, 128, 64), lambda i: (0, i, 0, 0)),
                   eb, eb, eb],
        out_shape=[jax.ShapeDtypeStruct((4, EP // 128, 128, 64), jnp.float32)]
                  + [jax.ShapeDtypeStruct((EP // 128, 128), jnp.float32)] * 3,
    )(rx2, ry2, rz2, W1r, b1r, W2, b2r, W3p, b3p)


def _p3_body(scomb_hbm, scalP_hbm, ux_hbm, uy_hbm, uz_hbm, snd_hbm, rcv_hbm,
             zeros_hbm, zidx_hbm,
             acc_hbm,
             acc_sh,
             snd0_v, snd1_v, rcv0_v, rcv1_v, su0_v, su1_v,
             ux0_v, ux1_v, uy0_v, uy1_v, uz0_v, uz1_v,
             feat0_v, feat1_v, idx0_v, idx1_v, rsc0_v, rsc1_v,
             rows0_v, rows1_v,
             sm_snd0, sm_snd1, sm_rcv0, sm_rcv1, sm_su0, sm_su1,
             sm_ux0, sm_ux1, sm_uy0, sm_uy1, sm_uz0, sm_uz1,
             sm_ft0, sm_ft1, sm_sc0, sm_sc1):
    core = lax.axis_index("c")
    sub = lax.axis_index("s")
    rows_per_sub = NP // NS
    per_sub = EP // NS
    nbatch = per_sub // B3
    snd_b = (snd0_v, snd1_v)
    rcv_b = (rcv0_v, rcv1_v)
    su_b = (su0_v, su1_v)
    ux_b = (ux0_v, ux1_v)
    uy_b = (uy0_v, uy1_v)
    uz_b = (uz0_v, uz1_v)
    feat_b = (feat0_v, feat1_v)
    idx_b = (idx0_v, idx1_v)
    sm_snd = (sm_snd0, sm_snd1)
    sm_rcv = (sm_rcv0, sm_rcv1)
    sm_su = (sm_su0, sm_su1)
    sm_ux = (sm_ux0, sm_ux1)
    sm_uy = (sm_uy0, sm_uy1)
    sm_uz = (sm_uz0, sm_uz1)
    sm_ft = (sm_ft0, sm_ft1)
    rows_b = (rows0_v, rows1_v)
    rsc_b = (rsc0_v, rsc1_v)
    sm_sc = (sm_sc0, sm_sc1)
    ROWS_BYTES = B3 * 128 * 4

    def e_start(sub_, t):
        return jnp.minimum(sub_ * per_sub + t * B3, EP - B3)

    def issue_in(chunk, t, s):
        e0 = e_start(sub, t)
        pltpu.async_copy(snd_hbm.at[pl.ds(e0, B3)], snd_b[s], sm_snd[s])
        pltpu.async_copy(rcv_hbm.at[pl.ds(e0, B3)], rcv_b[s], sm_rcv[s])
        pltpu.async_copy(scalP_hbm.at[chunk, pl.ds(e0, B3)], su_b[s], sm_su[s])
        pltpu.async_copy(ux_hbm.at[pl.ds(e0, B3)], ux_b[s], sm_ux[s])
        pltpu.async_copy(uy_hbm.at[pl.ds(e0, B3)], uy_b[s], sm_uy[s])
        pltpu.async_copy(uz_hbm.at[pl.ds(e0, B3)], uz_b[s], sm_uz[s])

    def wait_in(chunk, t, s, which):
        e0 = e_start(sub, t)
        if which == "snd":
            pltpu.make_async_copy(snd_hbm.at[pl.ds(e0, B3)], snd_b[s],
                                  sm_snd[s]).wait()
        elif which == "rcv":
            pltpu.make_async_copy(rcv_hbm.at[pl.ds(e0, B3)], rcv_b[s],
                                  sm_rcv[s]).wait()
        elif which == "u":
            pltpu.make_async_copy(ux_hbm.at[pl.ds(e0, B3)], ux_b[s],
                                  sm_ux[s]).wait()
            pltpu.make_async_copy(uy_hbm.at[pl.ds(e0, B3)], uy_b[s],
                                  sm_uy[s]).wait()
            pltpu.make_async_copy(uz_hbm.at[pl.ds(e0, B3)], uz_b[s],
                                  sm_uz[s]).wait()
        else:
            pltpu.make_async_copy(scalP_hbm.at[chunk, pl.ds(e0, B3)],
                                  su_b[s], sm_su[s]).wait()

    def issue_feat(chunk, t, s):
        wait_in(chunk, t, s, "snd")
        base = chunk * NP
        for g in range(B3 // 16):
            sl = pl.ds(g * 16, 16)
            idx_b[s][sl] = snd_b[s][sl] + base
        pltpu.async_copy(scomb_hbm.at[idx_b[s]], feat_b[s], sm_ft[s])

    def wait_feat(s):
        pltpu.make_async_copy(scomb_hbm.at[idx_b[s]], feat_b[s],
                              sm_ft[s]).wait()

    def compute_rows(chunk, t, s):
        wait_feat(s)
        wait_in(chunk, t, s, "su")
        wait_in(chunk, t, s, "u")
        su = su_b[s]
        feat = feat_b[s]
        rows_v = rows_b[s]
        ux_v, uy_v, uz_v = ux_b[s], uy_b[s], uz_b[s]

        @plsc.parallel_loop(0, B3, unroll=16)
        def _(ej):
            f0 = feat[ej, pl.ds(0, 16)]
            f1 = feat[ej, pl.ds(16, 16)]
            wfs0 = f0 * su[ej, pl.ds(0, 16)]
            wfs1 = f1 * su[ej, pl.ds(16, 16)]
            wfv0 = f0 * su[ej, pl.ds(32, 16)]
            wfv1 = f1 * su[ej, pl.ds(48, 16)]
            eidx = jnp.full((16,), ej, jnp.int32)
            bux = plsc.load_gather(ux_v, [eidx])
            buy = plsc.load_gather(uy_v, [eidx])
            buz = plsc.load_gather(uz_v, [eidx])
            rows_v[ej, pl.ds(0, 16)] = wfs0
            rows_v[ej, pl.ds(16, 16)] = wfs1
            rows_v[ej, pl.ds(32, 16)] = wfv0 * bux
            rows_v[ej, pl.ds(48, 16)] = wfv1 * bux
            rows_v[ej, pl.ds(64, 16)] = wfv0 * buy
            rows_v[ej, pl.ds(80, 16)] = wfv1 * buy
            rows_v[ej, pl.ds(96, 16)] = wfv0 * buz
            rows_v[ej, pl.ds(112, 16)] = wfv1 * buz
    def do_scatter(chunk, t, s):
        wait_in(chunk, t, s, "rcv")
        pltpu.make_async_copy(rows_b[1 - s], acc_sh.at[rsc_b[1 - s]],
                              sm_sc0).wait()
        for g in range(B3 // 16):
            sl = pl.ds(g * 16, 16)
            rsc_b[s][sl] = rcv_b[s][sl]
        pltpu.async_copy(rows_b[s], acc_sh.at[rsc_b[s]], sm_sc0, add=True)

    def zero_acc():
        for j in range(rows_per_sub // 64):
            pltpu.sync_copy(zeros_hbm, acc_sh.at[pl.ds(sub * rows_per_sub + j * 64, 64)])

    def chunk_body(k, _):
        chunk = core * 2 + k
        zero_acc()
        plsc.subcore_barrier()

        pltpu.sync_copy(zeros_hbm, rows1_v.at[pl.ds(0, 64)])
        pltpu.sync_copy(zeros_hbm.at[pl.ds(0, 16)], rows1_v.at[pl.ds(64, 16)])
        pltpu.sync_copy(zidx_hbm, rsc1_v)
        pltpu.async_copy(rows1_v, acc_sh.at[rsc1_v], sm_sc0, add=True)

        issue_in(chunk, 0, 0)
        issue_feat(chunk, 0, 0)

        def pair(q, _):
            for p in range(2):
                t = 2 * q + p
                issue_in(chunk, t + 1, 1 - p)
                compute_rows(chunk, t, p)
                issue_feat(chunk, t + 1, 1 - p)
                do_scatter(chunk, t, p)
            return ()

        lax.fori_loop(0, nbatch // 2, pair, ())
        wait_in(chunk, nbatch, 0, "rcv")
        wait_in(chunk, nbatch, 0, "su")
        wait_in(chunk, nbatch, 0, "u")
        wait_feat(0)
        pltpu.make_async_copy(rows_b[1], acc_sh.at[rsc_b[1]], sm_sc0).wait()

        plsc.subcore_barrier()
        r0 = sub * rows_per_sub
        pltpu.sync_copy(acc_sh.at[pl.ds(r0, rows_per_sub)],
                        acc_hbm.at[chunk, pl.ds(r0, rows_per_sub)])
        plsc.subcore_barrier()
        return ()

    lax.fori_loop(0, 2, chunk_body, ())


def _p3_call(s_comb, scalP, ux, uy, uz, snd, rcv):
    zeros = jnp.zeros((64, 128), jnp.float32)
    zidx = jnp.zeros((B3,), jnp.int32)
    return pl.kernel(
        _p3_body,
        out_type=jax.ShapeDtypeStruct((4, NP, 128), jnp.float32),
        mesh=plsc.VectorSubcoreMesh(core_axis_name="c", subcore_axis_name="s"),
        scratch_types=[
            pltpu.VMEM_SHARED((NP, 128), jnp.float32),
            pltpu.VMEM((B3,), jnp.int32),
            pltpu.VMEM((B3,), jnp.int32),
            pltpu.VMEM((B3,), jnp.int32),
            pltpu.VMEM((B3,), jnp.int32),
            pltpu.VMEM((B3, 64), jnp.float32),
            pltpu.VMEM((B3, 64), jnp.float32),
            pltpu.VMEM((B3,), jnp.float32),
            pltpu.VMEM((B3,), jnp.float32),
            pltpu.VMEM((B3,), jnp.float32),
            pltpu.VMEM((B3,), jnp.float32),
            pltpu.VMEM((B3,), jnp.float32),
            pltpu.VMEM((B3,), jnp.float32),
            pltpu.VMEM((B3, 32), jnp.float32),
            pltpu.VMEM((B3, 32), jnp.float32),
            pltpu.VMEM((B3,), jnp.int32),
            pltpu.VMEM((B3,), jnp.int32),
            pltpu.VMEM((B3,), jnp.int32),
            pltpu.VMEM((B3,), jnp.int32),
            pltpu.VMEM((B3, 128), jnp.float32),
            pltpu.VMEM((B3, 128), jnp.float32),
        ] + [pltpu.SemaphoreType.DMA] * 16,
        compiler_params=pltpu.CompilerParams(needs_layout_passes=False,
                                             use_tc_tiling_on_sc=False),
    )(s_comb, scalP, ux, uy, uz, snd, rcv, zeros, zidx)


def _p4_body(acc_ref, cnt_ref, s4_ref, Wgs_ref, Wgv_ref, Wos_ref, Wov_ref,
             Wrs_ref, Wrv_ref, out_ref):
    i = pl.program_id(0)
    cnt = jnp.maximum(cnt_ref[0, :, 0:1] + cnt_ref[1, :, 0:1], 1.0)
    inv = 1.0 / cnt
    acc = acc_ref[...]
    agg_s = jnp.concatenate([acc[c, :, 0:32] for c in range(4)], 1) * inv
    agg_vx = jnp.concatenate([acc[c, :, 32:64] for c in range(4)], 1) * inv
    agg_vy = jnp.concatenate([acc[c, :, 64:96] for c in range(4)], 1) * inv
    agg_vz = jnp.concatenate([acc[c, :, 96:128] for c in range(4)], 1) * inv
    exp_s = jnp.dot(agg_s, Wgs_ref[...], precision=_HIGH)
    act_s = jax.nn.gelu(exp_s[:, :2 * EMBED])
    gates = jax.nn.sigmoid(exp_s[:, 2 * EMBED:])
    Wgv = Wgv_ref[...]
    gvx = jnp.dot(agg_vx, Wgv, precision=_HIGH) * gates
    gvy = jnp.dot(agg_vy, Wgv, precision=_HIGH) * gates
    gvz = jnp.dot(agg_vz, Wgv, precision=_HIGH) * gates
    skip = jnp.concatenate([s4_ref[c] for c in range(4)], 1)
    cat = jnp.concatenate([act_s, skip], 1)
    out_s = jnp.dot(cat, Wos_ref[...], precision=_HIGH)
    Wov = Wov_ref[...]
    ovx = jnp.dot(gvx, Wov, precision=_HIGH)
    ovy = jnp.dot(gvy, Wov, precision=_HIGH)
    ovz = jnp.dot(gvz, Wov, precision=_HIGH)
    inv_v = ovx * ovx + ovy * ovy + ovz * ovz
    node = (jnp.dot(out_s, Wrs_ref[...], precision=_HIGH)
            + jnp.dot(inv_v, Wrv_ref[...], precision=_HIGH))
    rowid = i * 1024 + lax.broadcasted_iota(jnp.int32, (1024, 1), 0)
    node = jnp.where(rowid < N, node, 0.0)
    psum = jnp.sum(node, keepdims=True).reshape(1, 1)

    @pl.when(i == 0)
    def _():
        out_ref[...] = jnp.zeros((1, 1), jnp.float32)
    out_ref[...] += psum


def _p4_call(acc, cnt, s4, Wg_s, Wg_v, Wo_s, Wo_v, W_read_s, W_read_v):
    wf = lambda shape: pl.BlockSpec(shape, lambda i: tuple(0 for _ in shape))
    return pl.pallas_call(
        _p4_body,
        grid=(NP // 1024,),
        in_specs=[
            pl.BlockSpec((4, 1024, 128), lambda i: (0, i, 0)),
            pl.BlockSpec((2, 1024, 16), lambda i: (0, i, 0)),
            pl.BlockSpec((4, 1024, 32), lambda i: (0, i, 0)),
            wf((EMBED, 3 * EMBED)), wf((EMBED, EMBED)),
            wf((3 * EMBED, EMBED)), wf((EMBED, VEC_OUT)),
            wf((EMBED, 1)), wf((VEC_OUT, 1)),
        ],
        out_specs=pl.BlockSpec((1, 1), lambda i: (0, 0)),
        out_shape=jax.ShapeDtypeStruct((1, 1), jnp.float32),
    )(acc, cnt, s4, Wg_s, Wg_v, Wo_s, Wo_v, W_read_s, W_read_v)


@jax.jit
def kernel(atomic_numbers, positions, senders, receivers, embed_table,
           W1, b1, W2, b2, W3, b3, Wg_s, Wg_v, Wo_s, Wo_v,
           W_read_s, W_read_v):
    snd_p = jnp.concatenate([senders, jnp.zeros((EP - E,), jnp.int32)])
    rcv_p = jnp.concatenate([receivers, jnp.full((EP - E,), N, jnp.int32)])
    pos_flat = positions.reshape(-1)
    zf = jnp.concatenate([atomic_numbers.astype(jnp.float32),
                          jnp.zeros((NP - N,), jnp.float32)]).reshape(NP, 1)
    tab_pad = jnp.concatenate(
        [embed_table, jnp.zeros((EMBED - embed_table.shape[0], EMBED),
                                jnp.float32)], 0)

    s4 = _p2a_call(zf, tab_pad)
    s_comb = s4.reshape(4 * NP, 32)

    relx, rely, relz, cnt = _p1_call(pos_flat, snd_p, rcv_p)

    W3s, W3v = W3[:, :EMBED], W3[:, EMBED:]
    W3p = jnp.concatenate(
        [jnp.concatenate([W3s[:, 32 * c:32 * c + 32],
                          W3v[:, 32 * c:32 * c + 32]], 1) for c in range(4)], 1)
    b3s, b3v = b3[:EMBED], b3[EMBED:]
    b3p = jnp.concatenate(
        [jnp.concatenate([b3s[32 * c:32 * c + 32],
                          b3v[32 * c:32 * c + 32]]) for c in range(4)])
    scalP, ux2, uy2, uz2 = _p2b_call(
        relx.reshape(EP // 128, 128), rely.reshape(EP // 128, 128),
        relz.reshape(EP // 128, 128),
        W1, b1.reshape(1, HID), W2, b2.reshape(1, HID),
        W3p, b3p.reshape(1, 4 * HID))

    acc = _p3_call(s_comb, scalP.reshape(4, EP, 64), ux2.reshape(EP),
                   uy2.reshape(EP), uz2.reshape(EP), snd_p, rcv_p)

    total = _p4_call(acc, cnt, s4, Wg_s, Wg_v, Wo_s, Wo_v,
                     W_read_s, W_read_v)
    return total[0, 0] / N


# speedup vs baseline: 1.4252x; 1.2509x over previous
import functools
import jax
import jax.numpy as jnp
from jax import lax
from jax.experimental import pallas as pl
from jax.experimental.pallas import tpu as pltpu
from jax.experimental.pallas import tpu_sc as plsc

N = 10000
E = 160000
EMBED = 128
HID = 64
VEC_OUT = 64

NC = 2
NS = 16
EP = 163840
NP = 10240
B3 = 80
B1 = 512

_HIGH = jax.lax.Precision.HIGHEST


def _p1_body(pos_hbm, snd_hbm, rcv_hbm, zeros16_hbm, ones_hbm,
             relx_hbm, rely_hbm, relz_hbm, cnt_hbm,
             pos_v, snd_v, rcv_v, ox_v, oy_v, oz_v, ones_v, cnt_sh):
    core = lax.axis_index("c")
    sub = lax.axis_index("s")
    wid = sub * NC + core
    rows_per_sub = NP // NS
    pltpu.sync_copy(pos_hbm, pos_v)
    pltpu.sync_copy(ones_hbm, ones_v)
    for j in range(rows_per_sub // 64):
        pltpu.sync_copy(zeros16_hbm, cnt_sh.at[pl.ds(sub * rows_per_sub + j * 64, 64)])
    plsc.subcore_barrier()
    per_tile = EP // (NC * NS)

    def batch(b, _):
        e0 = wid * per_tile + b * B1
        pltpu.sync_copy(snd_hbm.at[pl.ds(e0, B1)], snd_v)
        pltpu.sync_copy(rcv_hbm.at[pl.ds(e0, B1)], rcv_v)
        for g in range(B1 // 16):
            sl = pl.ds(g * 16, 16)
            s3 = snd_v[sl] * 3
            r3 = rcv_v[sl] * 3
            for d, ref in ((0, ox_v), (1, oy_v), (2, oz_v)):
                ps = plsc.load_gather(pos_v, [s3 + d])
                pr = plsc.load_gather(pos_v, [r3 + d])
                ref[sl] = pr - ps
        pltpu.sync_copy(ox_v, relx_hbm.at[pl.ds(e0, B1)])
        pltpu.sync_copy(oy_v, rely_hbm.at[pl.ds(e0, B1)])
        pltpu.sync_copy(oz_v, relz_hbm.at[pl.ds(e0, B1)])
        pltpu.sync_copy(ones_v, cnt_sh.at[rcv_v], add=True)
        return ()

    lax.fori_loop(0, per_tile // B1, batch, ())
    plsc.subcore_barrier()
    r0 = sub * rows_per_sub
    pltpu.sync_copy(cnt_sh.at[pl.ds(r0, rows_per_sub)],
                    cnt_hbm.at[core, pl.ds(r0, rows_per_sub)])


def _p1_call(pos_flat, snd, rcv):
    zeros16 = jnp.zeros((64, 16), jnp.float32)
    ones = jnp.ones((B1, 16), jnp.float32)
    return pl.kernel(
        _p1_body,
        out_type=(jax.ShapeDtypeStruct((EP,), jnp.float32),) * 3
                 + (jax.ShapeDtypeStruct((2, NP, 16), jnp.float32),),
        mesh=plsc.VectorSubcoreMesh(core_axis_name="c", subcore_axis_name="s"),
        scratch_types=[
            pltpu.VMEM((N * 3,), jnp.float32),
            pltpu.VMEM((B1,), jnp.int32),
            pltpu.VMEM((B1,), jnp.int32),
            pltpu.VMEM((B1,), jnp.float32),
            pltpu.VMEM((B1,), jnp.float32),
            pltpu.VMEM((B1,), jnp.float32),
            pltpu.VMEM((B1, 16), jnp.float32),
            pltpu.VMEM_SHARED((NP, 16), jnp.float32),
        ],
        compiler_params=pltpu.CompilerParams(needs_layout_passes=False,
                                             use_tc_tiling_on_sc=False),
    )(pos_flat, snd, rcv, zeros16, ones)


def _p2a_body(zf_ref, tab_ref, s4_ref):
    zf = zf_ref[...]
    io = lax.broadcasted_iota(jnp.int32, (1, EMBED), 1).astype(jnp.float32)
    oh = (zf == io).astype(jnp.float32)
    s = jnp.dot(oh, tab_ref[...], precision=_HIGH)
    s4_ref[...] = jnp.stack([s[:, 32 * c:32 * c + 32] for c in range(4)], 0)


def _p2a_call(zf, tab_pad):
    return pl.pallas_call(
        _p2a_body,
        grid=(NP // 1024,),
        in_specs=[
            pl.BlockSpec((1024, 1), lambda i: (i, 0)),
            pl.BlockSpec((EMBED, EMBED), lambda i: (0, 0)),
        ],
        out_specs=pl.BlockSpec((4, 1024, 32), lambda i: (0, i, 0)),
        out_shape=jax.ShapeDtypeStruct((4, NP, 32), jnp.float32),
    )(zf, tab_pad)


def _p2b_body(rx_ref, ry_ref, rz_ref, W1r_ref, b1r_ref, W2_ref, b2r_ref,
              W3p_ref, b3p_ref, scalP_ref, ux_ref, uy_ref, uz_ref):
    rx, ry, rz = rx_ref[...], ry_ref[...], rz_ref[...]
    norm = jnp.sqrt(rx * rx + ry * ry + rz * rz)
    inv = 1.0 / jnp.maximum(norm, 1e-12)
    ux, uy, uz = rx * inv, ry * inv, rz * inv
    W1r, b1r = W1r_ref[...], b1r_ref[...]
    W2, b2r = W2_ref[...], b2r_ref[...]
    W3p, b3p = W3p_ref[...], b3p_ref[...]
    normT = norm.T
    slabs = []
    for r in range(8):
        nc = normT[:, r:r + 1]
        h = jax.nn.relu(nc * W1r + b1r)
        h = jax.nn.relu(jnp.dot(h, W2, precision=_HIGH) + b2r)
        scal = jnp.dot(h, W3p, precision=_HIGH) + b3p
        slabs.append(jnp.stack(
            [scal[:, 128 * p:128 * p + 128] for p in range(2)], 0))
    scalP_ref[...] = jnp.stack(slabs, axis=1)
    ux_ref[...], uy_ref[...], uz_ref[...] = ux, uy, uz


def _p2b_call(rx2, ry2, rz2, W1r, b1r, W2, b2r, W3p, b3p):
    eb = pl.BlockSpec((8, 128), lambda i: (i, 0))
    wf = lambda shape: pl.BlockSpec(shape, lambda i: tuple(0 for _ in shape))
    return pl.pallas_call(
        _p2b_body,
        grid=(EP // 1024,),
        in_specs=[eb, eb, eb,
                  wf((1, HID)), wf((1, HID)), wf((HID, HID)), wf((1, HID)),
                  wf((HID, 4 * HID)), wf((1, 4 * HID))],
        out_specs=[pl.BlockSpec((2, 8, 128, 128), lambda i: (0, i, 0, 0)),
                   eb, eb, eb],
        out_shape=[jax.ShapeDtypeStruct((2, EP // 128, 128, 128), jnp.float32)]
                  + [jax.ShapeDtypeStruct((EP // 128, 128), jnp.float32)] * 3,
    )(rx2, ry2, rz2, W1r, b1r, W2, b2r, W3p, b3p)


def _p3_body(scomb_hbm, scalP_hbm, ux_hbm, uy_hbm, uz_hbm, snd_hbm, rcv_hbm,
             zeros_hbm, zidx_hbm,
             acc_hbm,
             acc_sh,
             snd0_v, snd1_v, rcv0_v, rcv1_v, su0_v, su1_v,
             ux0_v, ux1_v, uy0_v, uy1_v, uz0_v, uz1_v,
             feat0_v, feat1_v, idx0_v, idx1_v, rsc0_v, rsc1_v,
             rows0_v, rows1_v,
             sm_snd0, sm_snd1, sm_rcv0, sm_rcv1, sm_su0, sm_su1,
             sm_ux0, sm_ux1, sm_uy0, sm_uy1, sm_uz0, sm_uz1,
             sm_ft0, sm_ft1, sm_sc0, sm_sc1):
    core = lax.axis_index("c")
    sub = lax.axis_index("s")
    rows_per_sub = NP // NS
    per_sub = EP // NS
    nbatch = per_sub // B3
    snd_b = (snd0_v, snd1_v)
    rcv_b = (rcv0_v, rcv1_v)
    su_b = (su0_v, su1_v)
    ux_b = (ux0_v, ux1_v)
    uy_b = (uy0_v, uy1_v)
    uz_b = (uz0_v, uz1_v)
    feat_b = (feat0_v, feat1_v)
    idx_b = (idx0_v, idx1_v)
    sm_snd = (sm_snd0, sm_snd1)
    sm_rcv = (sm_rcv0, sm_rcv1)
    sm_su = (sm_su0, sm_su1)
    sm_ux = (sm_ux0, sm_ux1)
    sm_uy = (sm_uy0, sm_uy1)
    sm_uz = (sm_uz0, sm_uz1)
    sm_ft = (sm_ft0, sm_ft1)
    rows_b = (rows0_v, rows1_v)
    rsc_b = (rsc0_v, rsc1_v)
    sm_sc = (sm_sc0, sm_sc1)
    ROWS_BYTES = B3 * 128 * 4

    def e_start(sub_, t):
        return jnp.minimum(sub_ * per_sub + t * B3, EP - B3)

    def issue_in(chunk, t, s):
        e0 = e_start(sub, t)
        pltpu.async_copy(snd_hbm.at[pl.ds(e0, B3)], snd_b[s], sm_snd[s])
        pltpu.async_copy(rcv_hbm.at[pl.ds(e0, B3)], rcv_b[s], sm_rcv[s])
        pltpu.async_copy(scalP_hbm.at[core, pl.ds(e0, B3)], su_b[s], sm_su[s])
        pltpu.async_copy(ux_hbm.at[pl.ds(e0, B3)], ux_b[s], sm_ux[s])
        pltpu.async_copy(uy_hbm.at[pl.ds(e0, B3)], uy_b[s], sm_uy[s])
        pltpu.async_copy(uz_hbm.at[pl.ds(e0, B3)], uz_b[s], sm_uz[s])

    def wait_in(chunk, t, s, which):
        e0 = e_start(sub, t)
        if which == "snd":
            pltpu.make_async_copy(snd_hbm.at[pl.ds(e0, B3)], snd_b[s],
                                  sm_snd[s]).wait()
        elif which == "rcv":
            pltpu.make_async_copy(rcv_hbm.at[pl.ds(e0, B3)], rcv_b[s],
                                  sm_rcv[s]).wait()
        elif which == "u":
            pltpu.make_async_copy(ux_hbm.at[pl.ds(e0, B3)], ux_b[s],
                                  sm_ux[s]).wait()
            pltpu.make_async_copy(uy_hbm.at[pl.ds(e0, B3)], uy_b[s],
                                  sm_uy[s]).wait()
            pltpu.make_async_copy(uz_hbm.at[pl.ds(e0, B3)], uz_b[s],
                                  sm_uz[s]).wait()
        else:
            pltpu.make_async_copy(scalP_hbm.at[core, pl.ds(e0, B3)],
                                  su_b[s], sm_su[s]).wait()

    def issue_feat(chunk, t, s):
        wait_in(chunk, t, s, "snd")
        base = chunk * NP
        for g in range(B3 // 16):
            sl = pl.ds(g * 16, 16)
            idx_b[s][sl] = snd_b[s][sl] + base
        pltpu.async_copy(scomb_hbm.at[idx_b[s]], feat_b[s], sm_ft[s])

    def wait_feat(s):
        pltpu.make_async_copy(scomb_hbm.at[idx_b[s]], feat_b[s],
                              sm_ft[s]).wait()

    def compute_rows(chunk, t, s, cb):
        wait_feat(s)
        wait_in(chunk, t, s, "su")
        wait_in(chunk, t, s, "u")
        su = su_b[s]
        feat = feat_b[s]
        rows_v = rows_b[s]
        ux_v, uy_v, uz_v = ux_b[s], uy_b[s], uz_b[s]

        @plsc.parallel_loop(0, B3, unroll=16)
        def _(ej):
            f0 = feat[ej, pl.ds(0, 16)]
            f1 = feat[ej, pl.ds(16, 16)]
            wfs0 = f0 * su[ej, pl.ds(cb, 16)]
            wfs1 = f1 * su[ej, pl.ds(cb + 16, 16)]
            wfv0 = f0 * su[ej, pl.ds(cb + 32, 16)]
            wfv1 = f1 * su[ej, pl.ds(cb + 48, 16)]
            eidx = jnp.full((16,), ej, jnp.int32)
            bux = plsc.load_gather(ux_v, [eidx])
            buy = plsc.load_gather(uy_v, [eidx])
            buz = plsc.load_gather(uz_v, [eidx])
            rows_v[ej, pl.ds(0, 16)] = wfs0
            rows_v[ej, pl.ds(16, 16)] = wfs1
            rows_v[ej, pl.ds(32, 16)] = wfv0 * bux
            rows_v[ej, pl.ds(48, 16)] = wfv1 * bux
            rows_v[ej, pl.ds(64, 16)] = wfv0 * buy
            rows_v[ej, pl.ds(80, 16)] = wfv1 * buy
            rows_v[ej, pl.ds(96, 16)] = wfv0 * buz
            rows_v[ej, pl.ds(112, 16)] = wfv1 * buz
    def do_scatter(chunk, t, s):
        wait_in(chunk, t, s, "rcv")
        pltpu.make_async_copy(rows_b[1 - s], acc_sh.at[rsc_b[1 - s]],
                              sm_sc0).wait()
        for g in range(B3 // 16):
            sl = pl.ds(g * 16, 16)
            rsc_b[s][sl] = rcv_b[s][sl]
        pltpu.async_copy(rows_b[s], acc_sh.at[rsc_b[s]], sm_sc0, add=True)

    def zero_acc():
        for j in range(rows_per_sub // 64):
            pltpu.sync_copy(zeros_hbm, acc_sh.at[pl.ds(sub * rows_per_sub + j * 64, 64)])

    for k in range(2):
        chunk = core * 2 + k
        zero_acc()
        plsc.subcore_barrier()

        pltpu.sync_copy(zeros_hbm, rows1_v.at[pl.ds(0, 64)])
        pltpu.sync_copy(zeros_hbm.at[pl.ds(0, 16)], rows1_v.at[pl.ds(64, 16)])
        pltpu.sync_copy(zidx_hbm, rsc1_v)
        pltpu.async_copy(rows1_v, acc_sh.at[rsc1_v], sm_sc0, add=True)

        issue_in(chunk, 0, 0)
        issue_feat(chunk, 0, 0)

        def pair(q, _, chunk=chunk, cb=64 * k):
            for p in range(2):
                t = 2 * q + p
                issue_in(chunk, t + 1, 1 - p)
                compute_rows(chunk, t, p, cb)
                issue_feat(chunk, t + 1, 1 - p)
                do_scatter(chunk, t, p)
            return ()

        lax.fori_loop(0, nbatch // 2, pair, ())
        wait_in(chunk, nbatch, 0, "rcv")
        wait_in(chunk, nbatch, 0, "su")
        wait_in(chunk, nbatch, 0, "u")
        wait_feat(0)
        pltpu.make_async_copy(rows_b[1], acc_sh.at[rsc_b[1]], sm_sc0).wait()

        plsc.subcore_barrier()
        r0 = sub * rows_per_sub
        pltpu.sync_copy(acc_sh.at[pl.ds(r0, rows_per_sub)],
                        acc_hbm.at[chunk, pl.ds(r0, rows_per_sub)])
        plsc.subcore_barrier()


def _p3_call(s_comb, scalP, ux, uy, uz, snd, rcv):
    zeros = jnp.zeros((64, 128), jnp.float32)
    zidx = jnp.zeros((B3,), jnp.int32)
    return pl.kernel(
        _p3_body,
        out_type=jax.ShapeDtypeStruct((4, NP, 128), jnp.float32),
        mesh=plsc.VectorSubcoreMesh(core_axis_name="c", subcore_axis_name="s"),
        scratch_types=[
            pltpu.VMEM_SHARED((NP, 128), jnp.float32),
            pltpu.VMEM((B3,), jnp.int32),
            pltpu.VMEM((B3,), jnp.int32),
            pltpu.VMEM((B3,), jnp.int32),
            pltpu.VMEM((B3,), jnp.int32),
            pltpu.VMEM((B3, 128), jnp.float32),
            pltpu.VMEM((B3, 128), jnp.float32),
            pltpu.VMEM((B3,), jnp.float32),
            pltpu.VMEM((B3,), jnp.float32),
            pltpu.VMEM((B3,), jnp.float32),
            pltpu.VMEM((B3,), jnp.float32),
            pltpu.VMEM((B3,), jnp.float32),
            pltpu.VMEM((B3,), jnp.float32),
            pltpu.VMEM((B3, 32), jnp.float32),
            pltpu.VMEM((B3, 32), jnp.float32),
            pltpu.VMEM((B3,), jnp.int32),
            pltpu.VMEM((B3,), jnp.int32),
            pltpu.VMEM((B3,), jnp.int32),
            pltpu.VMEM((B3,), jnp.int32),
            pltpu.VMEM((B3, 128), jnp.float32),
            pltpu.VMEM((B3, 128), jnp.float32),
        ] + [pltpu.SemaphoreType.DMA] * 16,
        compiler_params=pltpu.CompilerParams(needs_layout_passes=False,
                                             use_tc_tiling_on_sc=False),
    )(s_comb, scalP, ux, uy, uz, snd, rcv, zeros, zidx)


def _p4_body(acc_ref, cnt_ref, s4_ref, Wgs_ref, Wgv_ref, Wos_ref, Wov_ref,
             Wrs_ref, Wrv_ref, out_ref):
    i = pl.program_id(0)
    cnt = jnp.maximum(cnt_ref[0, :, 0:1] + cnt_ref[1, :, 0:1], 1.0)
    inv = 1.0 / cnt
    acc = acc_ref[...]
    agg_s = jnp.concatenate([acc[c, :, 0:32] for c in range(4)], 1) * inv
    agg_vx = jnp.concatenate([acc[c, :, 32:64] for c in range(4)], 1) * inv
    agg_vy = jnp.concatenate([acc[c, :, 64:96] for c in range(4)], 1) * inv
    agg_vz = jnp.concatenate([acc[c, :, 96:128] for c in range(4)], 1) * inv
    exp_s = jnp.dot(agg_s, Wgs_ref[...], precision=_HIGH)
    act_s = jax.nn.gelu(exp_s[:, :2 * EMBED])
    gates = jax.nn.sigmoid(exp_s[:, 2 * EMBED:])
    Wgv = Wgv_ref[...]
    gvx = jnp.dot(agg_vx, Wgv, precision=_HIGH) * gates
    gvy = jnp.dot(agg_vy, Wgv, precision=_HIGH) * gates
    gvz = jnp.dot(agg_vz, Wgv, precision=_HIGH) * gates
    skip = jnp.concatenate([s4_ref[c] for c in range(4)], 1)
    cat = jnp.concatenate([act_s, skip], 1)
    out_s = jnp.dot(cat, Wos_ref[...], precision=_HIGH)
    Wov = Wov_ref[...]
    ovx = jnp.dot(gvx, Wov, precision=_HIGH)
    ovy = jnp.dot(gvy, Wov, precision=_HIGH)
    ovz = jnp.dot(gvz, Wov, precision=_HIGH)
    inv_v = ovx * ovx + ovy * ovy + ovz * ovz
    node = (jnp.dot(out_s, Wrs_ref[...], precision=_HIGH)
            + jnp.dot(inv_v, Wrv_ref[...], precision=_HIGH))
    rowid = i * 1024 + lax.broadcasted_iota(jnp.int32, (1024, 1), 0)
    node = jnp.where(rowid < N, node, 0.0)
    psum = jnp.sum(node, keepdims=True).reshape(1, 1)

    @pl.when(i == 0)
    def _():
        out_ref[...] = jnp.zeros((1, 1), jnp.float32)
    out_ref[...] += psum


def _p4_call(acc, cnt, s4, Wg_s, Wg_v, Wo_s, Wo_v, W_read_s, W_read_v):
    wf = lambda shape: pl.BlockSpec(shape, lambda i: tuple(0 for _ in shape))
    return pl.pallas_call(
        _p4_body,
        grid=(NP // 1024,),
        in_specs=[
            pl.BlockSpec((4, 1024, 128), lambda i: (0, i, 0)),
            pl.BlockSpec((2, 1024, 16), lambda i: (0, i, 0)),
            pl.BlockSpec((4, 1024, 32), lambda i: (0, i, 0)),
            wf((EMBED, 3 * EMBED)), wf((EMBED, EMBED)),
            wf((3 * EMBED, EMBED)), wf((EMBED, VEC_OUT)),
            wf((EMBED, 1)), wf((VEC_OUT, 1)),
        ],
        out_specs=pl.BlockSpec((1, 1), lambda i: (0, 0)),
        out_shape=jax.ShapeDtypeStruct((1, 1), jnp.float32),
    )(acc, cnt, s4, Wg_s, Wg_v, Wo_s, Wo_v, W_read_s, W_read_v)


@jax.jit
def kernel(atomic_numbers, positions, senders, receivers, embed_table,
           W1, b1, W2, b2, W3, b3, Wg_s, Wg_v, Wo_s, Wo_v,
           W_read_s, W_read_v):
    snd_p = jnp.concatenate([senders, jnp.zeros((EP - E,), jnp.int32)])
    rcv_p = jnp.concatenate([receivers, jnp.full((EP - E,), N, jnp.int32)])
    pos_flat = positions.reshape(-1)
    zf = jnp.concatenate([atomic_numbers.astype(jnp.float32),
                          jnp.zeros((NP - N,), jnp.float32)]).reshape(NP, 1)
    tab_pad = jnp.concatenate(
        [embed_table, jnp.zeros((EMBED - embed_table.shape[0], EMBED),
                                jnp.float32)], 0)

    s4 = _p2a_call(zf, tab_pad)
    s_comb = s4.reshape(4 * NP, 32)

    relx, rely, relz, cnt = _p1_call(pos_flat, snd_p, rcv_p)

    W3s, W3v = W3[:, :EMBED], W3[:, EMBED:]
    W3p = jnp.concatenate(
        [jnp.concatenate([W3s[:, 32 * c:32 * c + 32],
                          W3v[:, 32 * c:32 * c + 32]], 1) for c in range(4)], 1)
    b3s, b3v = b3[:EMBED], b3[EMBED:]
    b3p = jnp.concatenate(
        [jnp.concatenate([b3s[32 * c:32 * c + 32],
                          b3v[32 * c:32 * c + 32]]) for c in range(4)])
    scalP, ux2, uy2, uz2 = _p2b_call(
        relx.reshape(EP // 128, 128), rely.reshape(EP // 128, 128),
        relz.reshape(EP // 128, 128),
        W1, b1.reshape(1, HID), W2, b2.reshape(1, HID),
        W3p, b3p.reshape(1, 4 * HID))

    acc = _p3_call(s_comb, scalP.reshape(2, EP, 128), ux2.reshape(EP),
                   uy2.reshape(EP), uz2.reshape(EP), snd_p, rcv_p)

    total = _p4_call(acc, cnt, s4, Wg_s, Wg_v, Wo_s, Wo_v,
                     W_read_s, W_read_v)
    return total[0, 0] / N
